# baseline (speedup 1.0000x reference)
import functools

import jax
import jax.numpy as jnp
from jax import lax
from jax.experimental import pallas as pl
from jax.experimental.pallas import tpu as pltpu
from jax.experimental.pallas import tpu_sc as plsc

N = 32
D = 128
M = 128
R = 4
SEQ = 8
GLOBALS = 50000
NT = 8
TILE = 6272
LAST = GLOBALS - (NT - 1) * TILE
B = SEQ * N
ROW = 416


def _sc_gather(table, batchinput):
    info = plsc.get_sparse_core_info()
    nw = info.num_cores * info.num_subcores
    bpw = B // nw
    wpt = N // bpw
    mesh = plsc.VectorSubcoreMesh(core_axis_name="c", subcore_axis_name="s")

    @functools.partial(
        pl.kernel,
        mesh=mesh,
        out_type=(
            jax.ShapeDtypeStruct((SEQ, N, D), jnp.float32),
            jax.ShapeDtypeStruct((SEQ, N * D), jnp.float32),
        ),
        scratch_types=[
            pltpu.VMEM((bpw,), jnp.int32),
            pltpu.VMEM((bpw, D), jnp.float32),
            pltpu.SemaphoreType.DMA,
        ],
    )
    def gather(bt_hbm, table_hbm, x3_hbm, xf_hbm, idx_v, rows_v, sem):
        wid = lax.axis_index("s") * info.num_cores + lax.axis_index("c")
        t = wid // wpt
        n0 = (wid % wpt) * bpw
        pltpu.sync_copy(bt_hbm.at[t, 0, pl.ds(n0, bpw)], idx_v)
        pltpu.async_copy(table_hbm.at[idx_v], rows_v, sem).wait()
        pltpu.sync_copy(rows_v, x3_hbm.at[t, pl.ds(n0, bpw)])
        for i in range(bpw):
            pltpu.sync_copy(rows_v.at[i],
                            xf_hbm.at[t, pl.ds((n0 + i) * D, D)])

    return gather(batchinput, table)


def _main_body(src_ref, dst_ref, et_ref, xg3_ref, xgf_ref, convw_ref, w0_ref,
               wg_ref, ug_ref, b_ref, w_hbm, out_ref, outs_ref, *scratch):
    wbufs = list(scratch[:NT])
    sems = scratch[NT]
    copies = []
    for i in range(NT):
        rows = TILE if i < NT - 1 else LAST
        cp = pltpu.make_async_copy(w_hbm.at[pl.ds(i * TILE, rows)],
                                   wbufs[i], sems.at[i])
        cp.start()
        copies.append(cp)

    xgm = xg3_ref[...].reshape(B, D)
    ng_all = jnp.dot(xgf_ref[...], wg_ref[...],
                     preferred_element_type=jnp.float32)
    prev_all = jnp.dot(xgm, w0_ref[...],
                       preferred_element_type=jnp.float32)
    xw = [jnp.dot(xgm, convw_ref[r], preferred_element_type=jnp.float32)
          for r in range(R)]

    iota_n = lax.broadcasted_iota(jnp.int32, (N, M), 0)
    iota_r = lax.broadcasted_iota(jnp.int32, (R, M), 0)
    eye_n = (lax.broadcasted_iota(jnp.int32, (N, N), 0) ==
             lax.broadcasted_iota(jnp.int32, (N, N), 1)).astype(jnp.float32)

    proposed = []
    for t in range(SEQ):
        src_row = src_ref[t:t + 1, :]
        dst_row = dst_ref[t:t + 1, :]
        et_row = et_ref[t:t + 1, :]
        oh_src = (src_row == iota_n).astype(jnp.float32)
        oh_dst = (dst_row == iota_n).astype(jnp.float32)
        mask4 = (et_row == iota_r).astype(jnp.float32)
        degs = lax.dot_general(oh_dst, mask4, (((1,), (1,)), ((), ())),
                               preferred_element_type=jnp.float32) + 1.0
        dis = lax.rsqrt(degs)
        invd = 1.0 / degs
        dis_s = lax.dot_general(dis, oh_src, (((0,), (0,)), ((), ())),
                                preferred_element_type=jnp.float32)
        dis_d = lax.dot_general(dis, oh_dst, (((0,), (0,)), ((), ())),
                                preferred_element_type=jnp.float32)
        norm4 = mask4 * dis_s * dis_d
        rhs = jnp.concatenate([norm4[r:r + 1, :] * oh_src for r in range(R)],
                              axis=0)
        adj = lax.dot_general(oh_dst, rhs, (((1,), (1,)), ((), ())),
                              preferred_element_type=jnp.float32)
        adj = adj + jnp.concatenate([invd[:, r:r + 1] * eye_n for r in range(R)],
                                    axis=1)
        xw_stack = jnp.concatenate([xw[r][t * N:(t + 1) * N, :]
                                    for r in range(R)], axis=0)
        conv = jnp.dot(adj, xw_stack, preferred_element_type=jnp.float32)
        proposed.append(conv + prev_all[t * N:(t + 1) * N, :])

    mem = jnp.zeros((N, D), jnp.float32)
    rows_out = []
    for t in range(SEQ):
        pg = jnp.dot(mem[0:1, :], ug_ref[...],
                     preferred_element_type=jnp.float32)
        u = 1.0 / (1.0 + jnp.exp(-(ng_all[t:t + 1, :] + pg)))
        mem = u * proposed[t] + (1.0 - u) * mem
        rows_out.append(jnp.maximum(mem[0:1, :], 0.0))
    x1 = jnp.concatenate(rows_out, axis=0)

    for i in range(NT):
        rows = TILE if i < NT - 1 else LAST
        copies[i].wait()
        lg = lax.dot_general(x1, wbufs[i][...], (((1,), (1,)), ((), ())),
                             preferred_element_type=jnp.float32)
        out_ref[:, i * TILE:i * TILE + rows] = (
            lg + b_ref[:, i * TILE:i * TILE + rows])

    lg = out_ref[...]
    m = jnp.max(lg, axis=1, keepdims=True)
    s = jnp.sum(jnp.exp(lg - m), axis=1, keepdims=True)
    out_ref[...] = lg - (m + jnp.log(s))
    outs_ref[...] = jnp.zeros((1, SEQ), jnp.float32)


def _run_main(src_all, dst_all, et_all, xg3, xgf, conv_W, W_0, wg, ug, b2d, w2g):
    wtypes = [pltpu.VMEM((TILE, D), jnp.float32) for _ in range(NT - 1)]
    wtypes.append(pltpu.VMEM((LAST, D), jnp.float32))
    return pl.pallas_call(
        _main_body,
        in_specs=[
            pl.BlockSpec((SEQ, M), lambda: (0, 0)),
            pl.BlockSpec((SEQ, M), lambda: (0, 0)),
            pl.BlockSpec((SEQ, M), lambda: (0, 0)),
            pl.BlockSpec((SEQ, N, D), lambda: (0, 0, 0)),
            pl.BlockSpec((SEQ, N * D), lambda: (0, 0)),
            pl.BlockSpec((R, D, D), lambda: (0, 0, 0)),
            pl.BlockSpec((D, D), lambda: (0, 0)),
            pl.BlockSpec((N * D, D), lambda: (0, 0)),
            pl.BlockSpec((D, D), lambda: (0, 0)),
            pl.BlockSpec((1, GLOBALS), lambda: (0, 0)),
            pl.BlockSpec(memory_space=pl.ANY),
        ],
        out_specs=[pl.BlockSpec((SEQ, GLOBALS), lambda: (0, 0)),
                   pl.BlockSpec((1, SEQ), lambda: (0, 0))],
        out_shape=[jax.ShapeDtypeStruct((SEQ, GLOBALS), jnp.float32),
                   jax.ShapeDtypeStruct((1, SEQ), jnp.float32)],
        scratch_shapes=wtypes + [pltpu.SemaphoreType.DMA((NT,))],
    )(src_all, dst_all, et_all, xg3, xgf, conv_W, W_0, wg, ug, b2d, w2g)


def kernel(batchinput_tensor, X, conv_W, W_0, update_gate_W, update_gate_U,
           lin2global_w, lin2global_b):
    packed = batchinput_tensor[:, 0, :]
    src_all = packed[:, N:N + M]
    dst_all = packed[:, N + M:N + 2 * M]
    et_all = packed[:, N + 2 * M:N + 3 * M]

    xg3, xgf = _sc_gather(X, batchinput_tensor)
    b2d = lin2global_b.reshape(1, GLOBALS)

    preds_g, preds_s = _run_main(src_all, dst_all, et_all, xg3, xgf, conv_W,
                                 W_0, update_gate_W, update_gate_U, b2d,
                                 lin2global_w)
    return preds_g, preds_s.reshape(SEQ)


# speedup vs baseline: 1.2506x; 1.2506x over previous
import functools

import jax
import jax.numpy as jnp
from jax import lax
from jax.experimental import pallas as pl
from jax.experimental.pallas import tpu as pltpu
from jax.experimental.pallas import tpu_sc as plsc

N = 32
D = 128
M = 128
R = 4
SEQ = 8
GLOBALS = 50000
NT = 4
TILE = 12544
LAST = GLOBALS - (NT - 1) * TILE
B = SEQ * N
ROW = 416


def _sc_gather(table, batchinput):
    info = plsc.get_sparse_core_info()
    nw = info.num_cores * info.num_subcores
    bpw = B // nw
    wpt = N // bpw
    mesh = plsc.VectorSubcoreMesh(core_axis_name="c", subcore_axis_name="s")

    @functools.partial(
        pl.kernel,
        mesh=mesh,
        out_type=(
            jax.ShapeDtypeStruct((SEQ, N, D), jnp.float32),
            jax.ShapeDtypeStruct((SEQ, N * D), jnp.float32),
        ),
        scratch_types=[
            pltpu.VMEM((bpw,), jnp.int32),
            pltpu.VMEM((bpw, D), jnp.float32),
            pltpu.SemaphoreType.DMA,
        ],
    )
    def gather(bt_hbm, table_hbm, x3_hbm, xf_hbm, idx_v, rows_v, sem):
        wid = lax.axis_index("s") * info.num_cores + lax.axis_index("c")
        t = wid // wpt
        n0 = (wid % wpt) * bpw
        pltpu.sync_copy(bt_hbm.at[t, 0, pl.ds(n0, bpw)], idx_v)
        pltpu.async_copy(table_hbm.at[idx_v], rows_v, sem).wait()
        pltpu.sync_copy(rows_v, x3_hbm.at[t, pl.ds(n0, bpw)])
        for i in range(bpw):
            pltpu.sync_copy(rows_v.at[i],
                            xf_hbm.at[t, pl.ds((n0 + i) * D, D)])

    return gather(batchinput, table)


def _main_body(src_ref, dst_ref, et_ref, xg3_ref, xgf_ref, convw_ref, w0_ref,
               wg_ref, ug_ref, b_ref, w_hbm, out_ref, outs_ref, *scratch):
    wbufs = list(scratch[:NT])
    sems = scratch[NT]
    copies = []
    for i in range(NT):
        rows = TILE if i < NT - 1 else LAST
        cp = pltpu.make_async_copy(w_hbm.at[pl.ds(i * TILE, rows)],
                                   wbufs[i], sems.at[i])
        cp.start()
        copies.append(cp)

    xgm = xg3_ref[...].reshape(B, D)
    ng_all = jnp.dot(xgf_ref[...], wg_ref[...],
                     preferred_element_type=jnp.float32)
    prev_all = jnp.dot(xgm, w0_ref[...],
                       preferred_element_type=jnp.float32)
    xw = [jnp.dot(xgm, convw_ref[r], preferred_element_type=jnp.float32)
          for r in range(R)]

    iota_n = lax.broadcasted_iota(jnp.int32, (N, M), 0)
    iota_r = lax.broadcasted_iota(jnp.int32, (R, M), 0)
    eye_n = (lax.broadcasted_iota(jnp.int32, (N, N), 0) ==
             lax.broadcasted_iota(jnp.int32, (N, N), 1)).astype(jnp.float32)

    proposed = []
    for t in range(SEQ):
        src_row = src_ref[t:t + 1, :]
        dst_row = dst_ref[t:t + 1, :]
        et_row = et_ref[t:t + 1, :]
        oh_src = (src_row == iota_n).astype(jnp.float32)
        oh_dst = (dst_row == iota_n).astype(jnp.float32)
        mask4 = (et_row == iota_r).astype(jnp.float32)
        degs = lax.dot_general(oh_dst, mask4, (((1,), (1,)), ((), ())),
                               preferred_element_type=jnp.float32) + 1.0
        dis = lax.rsqrt(degs)
        invd = 1.0 / degs
        dis_s = lax.dot_general(dis, oh_src, (((0,), (0,)), ((), ())),
                                preferred_element_type=jnp.float32)
        dis_d = lax.dot_general(dis, oh_dst, (((0,), (0,)), ((), ())),
                                preferred_element_type=jnp.float32)
        norm4 = mask4 * dis_s * dis_d
        rhs = jnp.concatenate([norm4[r:r + 1, :] * oh_src for r in range(R)],
                              axis=0)
        adj = lax.dot_general(oh_dst, rhs, (((1,), (1,)), ((), ())),
                              preferred_element_type=jnp.float32)
        adj = adj + jnp.concatenate([invd[:, r:r + 1] * eye_n for r in range(R)],
                                    axis=1)
        xw_stack = jnp.concatenate([xw[r][t * N:(t + 1) * N, :]
                                    for r in range(R)], axis=0)
        conv = jnp.dot(adj, xw_stack, preferred_element_type=jnp.float32)
        proposed.append(conv + prev_all[t * N:(t + 1) * N, :])

    mem = jnp.zeros((N, D), jnp.float32)
    rows_out = []
    for t in range(SEQ):
        pg = jnp.dot(mem[0:1, :], ug_ref[...],
                     preferred_element_type=jnp.float32)
        u = 1.0 / (1.0 + jnp.exp(-(ng_all[t:t + 1, :] + pg)))
        mem = u * proposed[t] + (1.0 - u) * mem
        rows_out.append(jnp.maximum(mem[0:1, :], 0.0))
    x1 = jnp.concatenate(rows_out, axis=0)

    for i in range(NT):
        rows = TILE if i < NT - 1 else LAST
        copies[i].wait()
        lg = lax.dot_general(x1, wbufs[i][...], (((1,), (1,)), ((), ())),
                             preferred_element_type=jnp.float32)
        out_ref[:, i * TILE:i * TILE + rows] = (
            lg + b_ref[:, i * TILE:i * TILE + rows])

    lg = out_ref[...]
    m = jnp.max(lg, axis=1, keepdims=True)
    s = jnp.sum(jnp.exp(lg - m), axis=1, keepdims=True)
    out_ref[...] = lg - (m + jnp.log(s))
    outs_ref[...] = jnp.zeros((1, SEQ), jnp.float32)


def _run_main(src_all, dst_all, et_all, xg3, xgf, conv_W, W_0, wg, ug, b2d, w2g):
    wtypes = [pltpu.VMEM((TILE, D), jnp.float32) for _ in range(NT - 1)]
    wtypes.append(pltpu.VMEM((LAST, D), jnp.float32))
    return pl.pallas_call(
        _main_body,
        in_specs=[
            pl.BlockSpec((SEQ, M), lambda: (0, 0)),
            pl.BlockSpec((SEQ, M), lambda: (0, 0)),
            pl.BlockSpec((SEQ, M), lambda: (0, 0)),
            pl.BlockSpec((SEQ, N, D), lambda: (0, 0, 0)),
            pl.BlockSpec((SEQ, N * D), lambda: (0, 0)),
            pl.BlockSpec((R, D, D), lambda: (0, 0, 0)),
            pl.BlockSpec((D, D), lambda: (0, 0)),
            pl.BlockSpec((N * D, D), lambda: (0, 0)),
            pl.BlockSpec((D, D), lambda: (0, 0)),
            pl.BlockSpec((1, GLOBALS), lambda: (0, 0)),
            pl.BlockSpec(memory_space=pl.ANY),
        ],
        out_specs=[pl.BlockSpec((SEQ, GLOBALS), lambda: (0, 0)),
                   pl.BlockSpec((1, SEQ), lambda: (0, 0))],
        out_shape=[jax.ShapeDtypeStruct((SEQ, GLOBALS), jnp.float32),
                   jax.ShapeDtypeStruct((1, SEQ), jnp.float32)],
        scratch_shapes=wtypes + [pltpu.SemaphoreType.DMA((NT,))],
    )(src_all, dst_all, et_all, xg3, xgf, conv_W, W_0, wg, ug, b2d, w2g)


def kernel(batchinput_tensor, X, conv_W, W_0, update_gate_W, update_gate_U,
           lin2global_w, lin2global_b):
    packed = batchinput_tensor[:, 0, :]
    src_all = packed[:, N:N + M]
    dst_all = packed[:, N + M:N + 2 * M]
    et_all = packed[:, N + 2 * M:N + 3 * M]

    xg3, xgf = _sc_gather(X, batchinput_tensor)
    b2d = lin2global_b.reshape(1, GLOBALS)

    preds_g, preds_s = _run_main(src_all, dst_all, et_all, xg3, xgf, conv_W,
                                 W_0, update_gate_W, update_gate_U, b2d,
                                 lin2global_w)
    return preds_g, preds_s.reshape(SEQ)


# speedup vs baseline: 1.2614x; 1.0086x over previous
import functools

import jax
import jax.numpy as jnp
from jax import lax
from jax.experimental import pallas as pl
from jax.experimental.pallas import tpu as pltpu
from jax.experimental.pallas import tpu_sc as plsc

N = 32
D = 128
M = 128
R = 4
SEQ = 8
GLOBALS = 50000
NT = 8
TILE = 6272
LAST = GLOBALS - (NT - 1) * TILE
B = SEQ * N
ROW = 416


def _sc_gather(table, batchinput):
    info = plsc.get_sparse_core_info()
    nw = info.num_cores * info.num_subcores
    bpw = B // nw
    wpt = N // bpw
    mesh = plsc.VectorSubcoreMesh(core_axis_name="c", subcore_axis_name="s")

    @functools.partial(
        pl.kernel,
        mesh=mesh,
        out_type=(
            jax.ShapeDtypeStruct((SEQ, N, D), jnp.float32),
            jax.ShapeDtypeStruct((SEQ, N * D), jnp.float32),
        ),
        scratch_types=[
            pltpu.VMEM((bpw,), jnp.int32),
            pltpu.VMEM((bpw, D), jnp.float32),
            pltpu.SemaphoreType.DMA,
        ],
    )
    def gather(bt_hbm, table_hbm, x3_hbm, xf_hbm, idx_v, rows_v, sem):
        wid = lax.axis_index("s") * info.num_cores + lax.axis_index("c")
        t = wid // wpt
        n0 = (wid % wpt) * bpw
        pltpu.sync_copy(bt_hbm.at[t, 0, pl.ds(n0, bpw)], idx_v)
        pltpu.async_copy(table_hbm.at[idx_v], rows_v, sem).wait()
        pltpu.sync_copy(rows_v, x3_hbm.at[t, pl.ds(n0, bpw)])
        for i in range(bpw):
            pltpu.sync_copy(rows_v.at[i],
                            xf_hbm.at[t, pl.ds((n0 + i) * D, D)])

    return gather(batchinput, table)


def _main_body(src_ref, dst_ref, et_ref, xg3_ref, xgf_ref, convw_ref, w0_ref,
               wg_ref, ug_ref, b_ref, w_hbm, out_ref, outs_ref, *scratch):
    wbufs = list(scratch[:NT])
    sems = scratch[NT]
    copies = []
    for i in range(NT):
        rows = TILE if i < NT - 1 else LAST
        cp = pltpu.make_async_copy(w_hbm.at[pl.ds(i * TILE, rows)],
                                   wbufs[i], sems.at[i])
        cp.start()
        copies.append(cp)

    xgm = xg3_ref[...].reshape(B, D)
    ng_all = jnp.dot(xgf_ref[...], wg_ref[...],
                     preferred_element_type=jnp.float32)
    prev_all = jnp.dot(xgm, w0_ref[...],
                       preferred_element_type=jnp.float32)
    xw = [jnp.dot(xgm, convw_ref[r], preferred_element_type=jnp.float32)
          for r in range(R)]

    iota_n = lax.broadcasted_iota(jnp.int32, (N, M), 0)
    iota_r = lax.broadcasted_iota(jnp.int32, (R, M), 0)
    eye_n = (lax.broadcasted_iota(jnp.int32, (N, N), 0) ==
             lax.broadcasted_iota(jnp.int32, (N, N), 1)).astype(jnp.float32)

    proposed = []
    for t in range(SEQ):
        src_row = src_ref[t:t + 1, :]
        dst_row = dst_ref[t:t + 1, :]
        et_row = et_ref[t:t + 1, :]
        oh_src = (src_row == iota_n).astype(jnp.float32)
        oh_dst = (dst_row == iota_n).astype(jnp.float32)
        mask4 = (et_row == iota_r).astype(jnp.float32)
        degs = lax.dot_general(oh_dst, mask4, (((1,), (1,)), ((), ())),
                               preferred_element_type=jnp.float32) + 1.0
        dis = lax.rsqrt(degs)
        invd = 1.0 / degs
        dis_s = lax.dot_general(dis, oh_src, (((0,), (0,)), ((), ())),
                                preferred_element_type=jnp.float32)
        dis_d = lax.dot_general(dis, oh_dst, (((0,), (0,)), ((), ())),
                                preferred_element_type=jnp.float32)
        norm4 = mask4 * dis_s * dis_d
        rhs = jnp.concatenate([norm4[r:r + 1, :] * oh_src for r in range(R)],
                              axis=0)
        adj = lax.dot_general(oh_dst, rhs, (((1,), (1,)), ((), ())),
                              preferred_element_type=jnp.float32)
        adj = adj + jnp.concatenate([invd[:, r:r + 1] * eye_n for r in range(R)],
                                    axis=1)
        xw_stack = jnp.concatenate([xw[r][t * N:(t + 1) * N, :]
                                    for r in range(R)], axis=0)
        conv = jnp.dot(adj, xw_stack, preferred_element_type=jnp.float32)
        proposed.append(conv + prev_all[t * N:(t + 1) * N, :])

    mem = jnp.zeros((N, D), jnp.float32)
    rows_out = []
    for t in range(SEQ):
        pg = jnp.dot(mem[0:1, :], ug_ref[...],
                     preferred_element_type=jnp.float32)
        u = 1.0 / (1.0 + jnp.exp(-(ng_all[t:t + 1, :] + pg)))
        mem = u * proposed[t] + (1.0 - u) * mem
        rows_out.append(jnp.maximum(mem[0:1, :], 0.0))
    x1 = jnp.concatenate(rows_out, axis=0)

    m_run = jnp.full((SEQ, 1), -1e30, jnp.float32)
    s_run = jnp.zeros((SEQ, 1), jnp.float32)
    for i in range(NT):
        rows = TILE if i < NT - 1 else LAST
        copies[i].wait()
        lg = lax.dot_general(x1, wbufs[i][...], (((1,), (1,)), ((), ())),
                             preferred_element_type=jnp.float32)
        lg = lg + b_ref[:, i * TILE:i * TILE + rows]
        out_ref[:, i * TILE:i * TILE + rows] = lg
        m_new = jnp.maximum(m_run, jnp.max(lg, axis=1, keepdims=True))
        s_run = (s_run * jnp.exp(m_run - m_new) +
                 jnp.sum(jnp.exp(lg - m_new), axis=1, keepdims=True))
        m_run = m_new

    out_ref[...] = out_ref[...] - (m_run + jnp.log(s_run))
    outs_ref[...] = jnp.zeros((1, SEQ), jnp.float32)


def _run_main(src_all, dst_all, et_all, xg3, xgf, conv_W, W_0, wg, ug, b2d, w2g):
    wtypes = [pltpu.VMEM((TILE, D), jnp.float32) for _ in range(NT - 1)]
    wtypes.append(pltpu.VMEM((LAST, D), jnp.float32))
    return pl.pallas_call(
        _main_body,
        in_specs=[
            pl.BlockSpec((SEQ, M), lambda: (0, 0)),
            pl.BlockSpec((SEQ, M), lambda: (0, 0)),
            pl.BlockSpec((SEQ, M), lambda: (0, 0)),
            pl.BlockSpec((SEQ, N, D), lambda: (0, 0, 0)),
            pl.BlockSpec((SEQ, N * D), lambda: (0, 0)),
            pl.BlockSpec((R, D, D), lambda: (0, 0, 0)),
            pl.BlockSpec((D, D), lambda: (0, 0)),
            pl.BlockSpec((N * D, D), lambda: (0, 0)),
            pl.BlockSpec((D, D), lambda: (0, 0)),
            pl.BlockSpec((1, GLOBALS), lambda: (0, 0)),
            pl.BlockSpec(memory_space=pl.ANY),
        ],
        out_specs=[pl.BlockSpec((SEQ, GLOBALS), lambda: (0, 0)),
                   pl.BlockSpec((1, SEQ), lambda: (0, 0))],
        out_shape=[jax.ShapeDtypeStruct((SEQ, GLOBALS), jnp.float32),
                   jax.ShapeDtypeStruct((1, SEQ), jnp.float32)],
        scratch_shapes=wtypes + [pltpu.SemaphoreType.DMA((NT,))],
    )(src_all, dst_all, et_all, xg3, xgf, conv_W, W_0, wg, ug, b2d, w2g)


def kernel(batchinput_tensor, X, conv_W, W_0, update_gate_W, update_gate_U,
           lin2global_w, lin2global_b):
    packed = batchinput_tensor[:, 0, :]
    src_all = packed[:, N:N + M]
    dst_all = packed[:, N + M:N + 2 * M]
    et_all = packed[:, N + 2 * M:N + 3 * M]

    xg3, xgf = _sc_gather(X, batchinput_tensor)
    b2d = lin2global_b.reshape(1, GLOBALS)

    preds_g, preds_s = _run_main(src_all, dst_all, et_all, xg3, xgf, conv_W,
                                 W_0, update_gate_W, update_gate_U, b2d,
                                 lin2global_w)
    return preds_g, preds_s.reshape(SEQ)
